# Initial kernel scaffold; baseline (speedup 1.0000x reference)
import jax
import jax.numpy as jnp
from jax.experimental import pallas as pl


def kernel(x):
    raise NotImplementedError("write your pallas kernel here")


# speedup vs baseline: 3.6835x; 3.6835x over previous
import functools

import jax
import jax.numpy as jnp
import numpy as np
from jax import lax
from jax.experimental import pallas as pl
from jax.experimental.pallas import tpu as pltpu
from jax.experimental.pallas import tpu_sc as plsc

B = 4096
F = 8192
K = 512
L = 16
NC, NS = 2, 16
NW = NC * NS
ROWS_PER_W = B // NW
NVR = F // L
KV = K // L
MARGIN = 0x000C0000
HI_INIT = np.int32(0x7F800001)

_mesh = plsc.VectorSubcoreMesh(
    core_axis_name="c", subcore_axis_name="s", num_cores=NC, num_subcores=NS
)


def _row_work(row, xb, ck, ci, ck2, ci2, sk0, si0, sk1, si1, ob, out_hbm, t_est):
    iota = lax.iota(jnp.int32, L)
    p1 = t_est + np.int32(MARGIN)
    p2 = jnp.maximum(t_est - np.int32(MARGIN), np.int32(-1))

    def pass_a(j, c):
        c1, c2 = c
        xv = xb[pl.ds(j * L, L)]
        key = lax.bitcast_convert_type(jnp.abs(xv), jnp.int32)
        idx = iota + j * L
        m1 = key > p1
        m2 = (key > p2) & jnp.logical_not(m1)
        c1c = jnp.minimum(c1, np.int32(K))
        plsc.store_compressed(sk0.at[pl.ds(c1c, L)], key, mask=m1)
        plsc.store_compressed(si0.at[pl.ds(c1c, L)], idx, mask=m1)
        plsc.store_compressed(ck.at[pl.ds(c2, L)], key, mask=m2)
        plsc.store_compressed(ci.at[pl.ds(c2, L)], idx, mask=m2)
        return (c1 + jnp.sum(m1.astype(jnp.int32)), c2 + jnp.sum(m2.astype(jnp.int32)))

    c1, nc = lax.fori_loop(0, NVR, pass_a, (np.int32(0), np.int32(0)))

    bad = (c1 > np.int32(K)) | (c1 + nc < np.int32(K))

    def _fallback(_):
        def fill(j, carry):
            xv = xb[pl.ds(j * L, L)]
            ck[pl.ds(j * L, L)] = lax.bitcast_convert_type(jnp.abs(xv), jnp.int32)
            ci[pl.ds(j * L, L)] = iota + j * L
            return carry

        lax.fori_loop(0, NVR, fill, np.int32(0))
        return np.int32(0), np.int32(F), np.int32(-1), HI_INIT

    def _keep(_):
        return c1, nc, p2, p1

    c1, nc, lo, hi = lax.cond(bad, _fallback, _keep, 0)
    krem = np.int32(K) - c1
    selo = c1

    def _cond(st):
        krem, nc, lo, hi, selo = st
        return (krem > np.int32(0)) & (nc > krem) & (hi > lo + np.int32(1))

    def _body(st):
        krem, nc, lo, hi, selo = st
        piv = lo + ((hi - lo) >> 1)
        nvr_c = (nc + np.int32(L - 1)) // jnp.int32(L)

        def cnt_body(j, acc):
            key = ck[pl.ds(j * L, L)]
            m = ((iota + j * L) < nc) & (key > piv)
            return acc + jnp.sum(m.astype(jnp.int32))

        cnt = lax.fori_loop(0, nvr_c, cnt_body, np.int32(0))
        take_hi = cnt >= krem

        def cpt_body(j, st2):
            co, so = st2
            key = ck[pl.ds(j * L, L)]
            idx = ci[pl.ds(j * L, L)]
            valid = (iota + j * L) < nc
            ones = valid & (key > piv)
            surv = jnp.where(take_hi, ones, valid & jnp.logical_not(ones))
            app = ones & jnp.logical_not(take_hi)
            plsc.store_compressed(ck2.at[pl.ds(co, L)], key, mask=surv)
            plsc.store_compressed(ci2.at[pl.ds(co, L)], idx, mask=surv)
            plsc.store_compressed(sk0.at[pl.ds(so, L)], key, mask=app)
            plsc.store_compressed(si0.at[pl.ds(so, L)], idx, mask=app)
            return (co + jnp.sum(surv.astype(jnp.int32)),
                    so + jnp.sum(app.astype(jnp.int32)))

        co, so = lax.fori_loop(0, nvr_c, cpt_body, (np.int32(0), selo))

        def cpy_body(j, carry):
            ck[pl.ds(j * L, L)] = ck2[pl.ds(j * L, L)]
            ci[pl.ds(j * L, L)] = ci2[pl.ds(j * L, L)]
            return carry

        nvr_o = (co + np.int32(L - 1)) // jnp.int32(L)
        lax.fori_loop(0, nvr_o, cpy_body, np.int32(0))

        new_lo = jnp.where(take_hi, piv, lo)
        new_hi = jnp.where(take_hi, hi, piv)
        new_krem = jnp.where(take_hi, krem, krem - cnt)
        new_nc = jnp.where(take_hi, cnt, nc - cnt)
        return (new_krem, new_nc, new_lo, new_hi, so)

    krem, nc, lo, hi, selo = lax.while_loop(
        _cond, _body, (krem, nc, lo, hi, selo)
    )

    def fin_body(j, carry):
        key = ck[pl.ds(j * L, L)]
        idx = ci[pl.ds(j * L, L)]
        gpos = iota + j * L
        m = gpos < krem
        so = selo + j * L
        plsc.store_compressed(sk0.at[pl.ds(so, L)], key, mask=m)
        plsc.store_compressed(si0.at[pl.ds(so, L)], idx, mask=m)
        return carry

    lax.fori_loop(0, (krem + np.int32(L - 1)) // jnp.int32(L), fin_body, 0)
    t_new = lo + ((hi - lo) >> 1)

    def vsort_pass(kr, ir):
        def vb(v, carry):
            o = v * L
            sk, si = plsc.sort_key_val(kr[pl.ds(o, L)], ir[pl.ds(o, L)],
                                       descending=True)
            kr[pl.ds(o, L)] = sk
            ir[pl.ds(o, L)] = si
            return carry

        lax.fori_loop(0, KV, vb, 0)

    def split_stage(r, src_k, src_i, dst_k, dst_i):
        def sb(q, carry):
            m = q // r
            t = q % r
            base = 2 * m * r
            oa = (base + t) * L
            obv = (base + r + (r - 1 - t)) * L
            ka = src_k[pl.ds(oa, L)]
            ia = src_i[pl.ds(oa, L)]
            kb = lax.rev(src_k[pl.ds(obv, L)], (0,))
            ib = lax.rev(src_i[pl.ds(obv, L)], (0,))
            pred = (ka > kb) | ((ka == kb) & (ia < ib))
            dst_k[pl.ds(oa, L)] = jnp.where(pred, ka, kb)
            dst_i[pl.ds(oa, L)] = jnp.where(pred, ia, ib)
            ol = (base + r + t) * L
            dst_k[pl.ds(ol, L)] = jnp.where(pred, kb, ka)
            dst_i[pl.ds(ol, L)] = jnp.where(pred, ib, ia)
            return carry

        lax.fori_loop(0, KV // 2, sb, 0)

    def plain_stage(d, kr, ir):
        def pb(q, carry):
            blk = q // d
            t = q % d
            pos = (blk * 2 * d + t) * L
            pd = d * L
            ka = kr[pl.ds(pos, L)]
            ia = ir[pl.ds(pos, L)]
            kb = kr[pl.ds(pos + pd, L)]
            ib = ir[pl.ds(pos + pd, L)]
            pred = (ka > kb) | ((ka == kb) & (ia < ib))
            kr[pl.ds(pos, L)] = jnp.where(pred, ka, kb)
            ir[pl.ds(pos, L)] = jnp.where(pred, ia, ib)
            kr[pl.ds(pos + pd, L)] = jnp.where(pred, kb, ka)
            ir[pl.ds(pos + pd, L)] = jnp.where(pred, ib, ia)
            return carry

        lax.fori_loop(0, KV // 2, pb, 0)

    vsort_pass(sk0, si0)
    bufs = [(sk0, si0), (sk1, si1)]
    cur = 0
    for r in (1, 2, 4, 8, 16):
        split_stage(r, bufs[cur][0], bufs[cur][1], bufs[1 - cur][0], bufs[1 - cur][1])
        cur ^= 1
        d = r // 2
        while d >= 1:
            plain_stage(d, bufs[cur][0], bufs[cur][1])
            d //= 2
        vsort_pass(bufs[cur][0], bufs[cur][1])
    fk, fi = bufs[cur]


    def out_body(v, carry):
        o = v * L
        chan0 = iota * 0
        idx = fi[pl.ds(o, L)]
        val = plsc.load_gather(xb, [idx])
        pos = iota + o
        rank = pos.astype(jnp.float32) / np.float32(K - 1)
        drop = (val == np.float32(0.0)).astype(jnp.float32)
        plsc.store_scatter(ob, [pos, chan0], val)
        plsc.store_scatter(ob, [pos, chan0 + 1], rank)
        plsc.store_scatter(ob, [pos, chan0 + 2], drop)
        return carry

    lax.fori_loop(0, KV, out_body, 0)
    pltpu.sync_copy(ob, out_hbm.at[row])
    return t_new


@functools.partial(
    pl.kernel,
    out_type=jax.ShapeDtypeStruct((B, K, 3), jnp.float32),
    mesh=_mesh,
    compiler_params=pltpu.CompilerParams(needs_layout_passes=False),
    scratch_types=[
        pltpu.VMEM((F,), jnp.float32),
        pltpu.VMEM((F,), jnp.float32),
        pltpu.VMEM((F + L,), jnp.int32),
        pltpu.VMEM((F + L,), jnp.int32),
        pltpu.VMEM((F + L,), jnp.int32),
        pltpu.VMEM((F + L,), jnp.int32),
        pltpu.VMEM((K + L,), jnp.int32),
        pltpu.VMEM((K + L,), jnp.int32),
        pltpu.VMEM((K + L,), jnp.int32),
        pltpu.VMEM((K + L,), jnp.int32),
        pltpu.VMEM((K, 3), jnp.float32),
        pltpu.SemaphoreType.DMA,
        pltpu.SemaphoreType.DMA,
    ],
)
def _topk_tokens(x_hbm, out_hbm, xb0, xb1, ck, ci, ck2, ci2,
                 sk0, si0, sk1, si1, ob, sx0, sx1):
    wid = lax.axis_index("s") * NC + lax.axis_index("c")
    row0 = wid * ROWS_PER_W

    pltpu.async_copy(x_hbm.at[row0], xb0, sx0)
    pltpu.async_copy(x_hbm.at[row0 + 1], xb1, sx1)

    def it(i, t_est):
        r0 = row0 + 2 * i
        pltpu.make_async_copy(x_hbm.at[0], xb0, sx0).wait()
        t_est = _row_work(r0, xb0, ck, ci, ck2, ci2, sk0, si0, sk1, si1, ob,
                          out_hbm, t_est)
        pltpu.async_copy(x_hbm.at[jnp.minimum(r0 + 2, B - 1)], xb0, sx0)
        pltpu.make_async_copy(x_hbm.at[0], xb1, sx1).wait()
        t_est = _row_work(r0 + 1, xb1, ck, ci, ck2, ci2, sk0, si0, sk1, si1, ob,
                          out_hbm, t_est)
        pltpu.async_copy(x_hbm.at[jnp.minimum(r0 + 3, B - 1)], xb1, sx1)
        return t_est

    lax.fori_loop(0, ROWS_PER_W // 2, it, np.int32(0x3FF00000))

    pltpu.make_async_copy(x_hbm.at[0], xb0, sx0).wait()
    pltpu.make_async_copy(x_hbm.at[0], xb1, sx1).wait()


def kernel(x):
    return _topk_tokens(x)
